# Initial kernel scaffold; baseline (speedup 1.0000x reference)
import jax
import jax.numpy as jnp
from jax.experimental import pallas as pl


def kernel(nodes, edges, edge_index, W_e, b_e, W_n, b_n):
    raise NotImplementedError("write your pallas kernel here")


# speedup vs baseline: 2.2692x; 2.2692x over previous
import functools

import jax
import jax.numpy as jnp
from jax import lax
from jax.experimental import pallas as pl
from jax.experimental.pallas import tpu as pltpu
from jax.experimental.pallas import tpu_sc as plsc

NCORES = 2
NSUB = 16
LANES = 16


def _proj_nodes_kernel(nodes_ref, w1_ref, w2_ref, pn_ref, pd_ref):
    nb = nodes_ref[...]
    pn_ref[...] = jnp.dot(nb, w1_ref[...], preferred_element_type=jnp.float32)
    pd_ref[...] = jnp.dot(nb, w2_ref[...], preferred_element_type=jnp.float32)


def _proj_edges_kernel(edges_ref, w3_ref, be_ref, epj_ref):
    epj_ref[...] = (
        jnp.dot(edges_ref[...], w3_ref[...], preferred_element_type=jnp.float32)
        + be_ref[...][None, :]
    )


def _node_update_kernel(nodes_ref, a0_ref, a1_ref, wn_ref, bn_ref, out_ref):
    nb = nodes_ref[...]
    wn = wn_ref[...]
    d = nb.shape[1]
    h = a0_ref.shape[1]
    acc = jnp.dot(nb, wn[:d], preferred_element_type=jnp.float32)
    acc = acc + jnp.dot(a0_ref[...], wn[d:d + h], preferred_element_type=jnp.float32)
    acc = acc + jnp.dot(a1_ref[...], wn[d + h:], preferred_element_type=jnp.float32)
    out_ref[...] = jnp.maximum(acc + bn_ref[...][None, :], 0.0)


def _make_sc_edge_kernel(n_nodes, n_edges, feat_half, chunk):
    N, E, H, CH = n_nodes, n_edges, feat_half, chunk
    ept = E // NSUB
    nch = ept // CH
    rpt = 1000
    ntl = N // rpt

    mesh = plsc.VectorSubcoreMesh(
        core_axis_name="c", subcore_axis_name="s",
        num_cores=NCORES, num_subcores=NSUB)

    @functools.partial(
        pl.kernel,
        out_type=[
            jax.ShapeDtypeStruct((E, 2 * H), jnp.float32),
            jax.ShapeDtypeStruct((NCORES * N, H), jnp.float32),
        ],
        mesh=mesh,
        scratch_types=[
            pltpu.VMEM((CH,), jnp.int32),
            pltpu.VMEM((CH,), jnp.int32),
            pltpu.VMEM((CH,), jnp.int32),
            pltpu.VMEM((CH,), jnp.int32),
            pltpu.VMEM((CH, H), jnp.float32),
            pltpu.VMEM((CH, H), jnp.float32),
            pltpu.VMEM((CH, H), jnp.float32),
            pltpu.VMEM_SHARED((N, H), jnp.float32),
            pltpu.SemaphoreType.DMA,
        ],
    )
    def sc_edge(pn_hbm, pd_hbm, epj_hbm, src_hbm, dst_hbm, zero_hbm,
                enew_hbm, agg_hbm,
                idx_s, idx_d, idx_gs, idx_gd, buf_a, buf_b, buf_c,
                agg_sh, sem):
        c = lax.axis_index("c")
        s = lax.axis_index("s")
        cn = c * N

        @pl.when(s < ntl)
        def _init():
            pltpu.sync_copy(zero_hbm.at[pl.ds(s * rpt, rpt)],
                            agg_sh.at[pl.ds(s * rpt, rpt)])
        plsc.subcore_barrier()

        def chunk_body(k, carry):
            base = s * ept + k * CH
            pltpu.sync_copy(src_hbm.at[pl.ds(base, CH)], idx_s)
            pltpu.sync_copy(dst_hbm.at[pl.ds(base, CH)], idx_d)
            for g in range(CH // LANES):
                sl = pl.ds(g * LANES, LANES)
                idx_gs[sl] = idx_s[sl] + cn
                idx_gd[sl] = idx_d[sl] + cn
            d1 = pltpu.make_async_copy(pn_hbm.at[idx_gs], buf_a, sem)
            d2 = pltpu.make_async_copy(pd_hbm.at[idx_gd], buf_b, sem)
            d3 = pltpu.make_async_copy(
                epj_hbm.at[pl.ds(c * E + base, CH)], buf_c, sem)
            d1.start()
            d2.start()
            d3.start()
            d1.wait()
            d2.wait()
            d3.wait()

            def row_body(r, rcarry):
                for j in range(H // LANES):
                    sl = pl.ds(j * LANES, LANES)
                    v = buf_a[r, sl] + buf_b[r, sl] + buf_c[r, sl]
                    buf_a[r, sl] = jnp.maximum(v, 0.0)
                return rcarry

            lax.fori_loop(0, CH, row_body, 0)

            pltpu.sync_copy(buf_a,
                            enew_hbm.at[pl.ds(base, CH), pl.ds(c * H, H)])
            pltpu.sync_copy(buf_a, agg_sh.at[idx_d], add=True)
            return carry

        lax.fori_loop(0, nch, chunk_body, 0)
        plsc.subcore_barrier()

        @pl.when(s < ntl)
        def _writeback():
            pltpu.sync_copy(agg_sh.at[pl.ds(s * rpt, rpt)],
                            agg_hbm.at[pl.ds(cn + s * rpt, rpt)])

    return sc_edge


def kernel(nodes, edges, edge_index, W_e, b_e, W_n, b_n):
    N, D = nodes.shape
    E, De = edges.shape
    H = D // 2

    W1 = W_e[:D]
    W2 = W_e[D:2 * D]
    W3 = W_e[2 * D:]
    src = edge_index[0]
    dst = edge_index[1]

    BN = 2000
    NB = N // BN
    pn2, pd2 = pl.pallas_call(
        _proj_nodes_kernel,
        grid=(2, NB),
        in_specs=[
            pl.BlockSpec((BN, D), lambda h, i: (i, 0)),
            pl.BlockSpec((D, H), lambda h, i: (0, h)),
            pl.BlockSpec((D, H), lambda h, i: (0, h)),
        ],
        out_specs=[
            pl.BlockSpec((BN, H), lambda h, i: (h * NB + i, 0)),
            pl.BlockSpec((BN, H), lambda h, i: (h * NB + i, 0)),
        ],
        out_shape=[
            jax.ShapeDtypeStruct((2 * N, H), jnp.float32),
            jax.ShapeDtypeStruct((2 * N, H), jnp.float32),
        ],
    )(nodes, W1, W2)

    BE = 8000
    NEB = E // BE
    epj2 = pl.pallas_call(
        _proj_edges_kernel,
        grid=(2, NEB),
        in_specs=[
            pl.BlockSpec((BE, De), lambda h, i: (i, 0)),
            pl.BlockSpec((De, H), lambda h, i: (0, h)),
            pl.BlockSpec((H,), lambda h, i: (h,)),
        ],
        out_specs=pl.BlockSpec((BE, H), lambda h, i: (h * NEB + i, 0)),
        out_shape=jax.ShapeDtypeStruct((2 * E, H), jnp.float32),
    )(edges, W3, b_e)

    zeros = jnp.zeros((2 * N, H), jnp.float32)
    sc_edge = _make_sc_edge_kernel(N, E, H, 80)
    enew, agg2 = sc_edge(pn2, pd2, epj2, src, dst, zeros)

    BN2 = 2000
    NB2 = N // BN2
    n_new = pl.pallas_call(
        _node_update_kernel,
        grid=(NB2,),
        in_specs=[
            pl.BlockSpec((BN2, D), lambda i: (i, 0)),
            pl.BlockSpec((BN2, H), lambda i: (i, 0)),
            pl.BlockSpec((BN2, H), lambda i: (NB2 + i, 0)),
            pl.BlockSpec((2 * D, D), lambda i: (0, 0)),
            pl.BlockSpec((D,), lambda i: (0,)),
        ],
        out_specs=pl.BlockSpec((BN2, D), lambda i: (i, 0)),
        out_shape=jax.ShapeDtypeStruct((N, D), jnp.float32),
    )(nodes, agg2, agg2, W_n, b_n)

    return (n_new, enew)
